# Initial kernel scaffold; baseline (speedup 1.0000x reference)
import jax
import jax.numpy as jnp
from jax.experimental import pallas as pl


def kernel(word, entity, attribute_key, Wword, Went, Wattr, gamma, beta):
    raise NotImplementedError("write your pallas kernel here")


# speedup vs baseline: 1.3753x; 1.3753x over previous
import functools

import jax
import jax.numpy as jnp
from jax import lax
from jax.experimental import pallas as pl
from jax.experimental.pallas import tpu as pltpu
from jax.experimental.pallas import tpu_sc as plsc

VOCAB = 1000000
POS = 512
DIM = 32
EPS = 1e-5

NC, NS, LANES = 2, 16, 16
NW = NC * NS

CHUNK = 128


def _rsqrt(x):
    i = plsc.bitcast(x, jnp.int32)
    i = jnp.int32(0x5F3759DF) - jnp.right_shift(i, 1)
    y = plsc.bitcast(i, jnp.float32)
    xh = x * 0.5
    for _ in range(3):
        y = y * (1.5 - xh * y * y)
    return y


def _body(widx_hbm, eidx_hbm, aidx_hbm, wword_hbm, went_hbm, wattr_hbm,
          gb_hbm, bb_hbm, out_hbm,
          went_v, wattr_v, gb_v, bb_v, widx_v, eidx_v, aidx_v, rows_v, sem,
          *, n_tokens):
    per_w = n_tokens // NW
    n_chunks = per_w // CHUNK
    wid = lax.axis_index("s") * NC + lax.axis_index("c")

    pltpu.sync_copy(went_hbm, went_v)
    pltpu.sync_copy(wattr_hbm, wattr_v)
    pltpu.sync_copy(gb_hbm, gb_v)
    pltpu.sync_copy(bb_hbm, bb_v)

    iota = lax.iota(jnp.int32, LANES)

    def chunk_body(ci):
        off = wid * per_w + ci * CHUNK
        pltpu.sync_copy(widx_hbm.at[pl.ds(off, CHUNK)], widx_v)
        pltpu.sync_copy(eidx_hbm.at[pl.ds(off, CHUNK)], eidx_v)
        pltpu.sync_copy(aidx_hbm.at[pl.ds(off, CHUNK)], aidx_v)
        pltpu.async_copy(wword_hbm.at[widx_v], rows_v, sem).wait()

        def group_body(g):
            tvec = g * LANES + iota
            evec = eidx_v[pl.ds(g * LANES, LANES)]
            avec = aidx_v[pl.ds(g * LANES, LANES)]
            zero = jnp.zeros((LANES,), jnp.float32)
            ssum = zero
            ssq = zero
            for d in range(DIM):
                dsp = jnp.full((LANES,), d, jnp.int32)
                wv = plsc.load_gather(rows_v, [tvec, dsp])
                ev = plsc.load_gather(went_v, [evec, dsp])
                av = plsc.load_gather(wattr_v, [avec, dsp])
                s = wv + ev + av
                ssum = ssum + s
                ssq = ssq + s * s
                plsc.store_scatter(rows_v, [tvec, dsp], s)
            mean = ssum * (1.0 / DIM)
            var = ssq * (1.0 / DIM) - mean * mean
            rstd = _rsqrt(var + EPS)
            for d in range(DIM):
                dsp = jnp.full((LANES,), d, jnp.int32)
                s = plsc.load_gather(rows_v, [tvec, dsp])
                gv = gb_v[d, :]
                bv = bb_v[d, :]
                a = rstd * gv
                o = (s - mean) * a + bv
                plsc.store_scatter(rows_v, [tvec, dsp], o)

        pl.loop(0, CHUNK // LANES)(group_body)
        pltpu.sync_copy(rows_v, out_hbm.at[pl.ds(off, CHUNK)])

    pl.loop(0, n_chunks)(chunk_body)


def kernel(word, entity, attribute_key, Wword, Went, Wattr, gamma, beta):
    shape = word.shape
    n_tokens = shape[0] * shape[1]
    assert n_tokens % (NW * CHUNK) == 0

    widx = word.reshape(n_tokens).astype(jnp.int32)
    eidx = entity.reshape(n_tokens).astype(jnp.int32)
    aidx = attribute_key.reshape(n_tokens).astype(jnp.int32)
    gb = jnp.broadcast_to(gamma.astype(jnp.float32)[:, None], (DIM, LANES))
    bb = jnp.broadcast_to(beta.astype(jnp.float32)[:, None], (DIM, LANES))

    mesh = plsc.VectorSubcoreMesh(core_axis_name="c", subcore_axis_name="s")
    run = pl.kernel(
        functools.partial(_body, n_tokens=n_tokens),
        out_type=jax.ShapeDtypeStruct((n_tokens, DIM), jnp.float32),
        mesh=mesh,
        compiler_params=pltpu.CompilerParams(
            needs_layout_passes=False, use_tc_tiling_on_sc=False),
        scratch_types=[
            pltpu.VMEM((POS, DIM), jnp.float32),
            pltpu.VMEM((POS, DIM), jnp.float32),
            pltpu.VMEM((DIM, LANES), jnp.float32),
            pltpu.VMEM((DIM, LANES), jnp.float32),
            pltpu.VMEM((CHUNK,), jnp.int32),
            pltpu.VMEM((CHUNK,), jnp.int32),
            pltpu.VMEM((CHUNK,), jnp.int32),
            pltpu.VMEM((CHUNK, DIM), jnp.float32),
            pltpu.SemaphoreType.DMA,
        ],
    )
    out = run(widx, eidx, aidx,
              Wword.astype(jnp.float32), Went.astype(jnp.float32),
              Wattr.astype(jnp.float32), gb, bb)
    return out.reshape(shape[0], shape[1], DIM)


# speedup vs baseline: 2.1902x; 1.5925x over previous
import functools

import jax
import jax.numpy as jnp
from jax import lax
from jax.experimental import pallas as pl
from jax.experimental.pallas import tpu as pltpu
from jax.experimental.pallas import tpu_sc as plsc

VOCAB = 1000000
POS = 512
DIM = 32
EPS = 1e-5

NC, NS, LANES = 2, 16, 16
NW = NC * NS

CHUNK = 128


def _rsqrt(x):
    i = plsc.bitcast(x, jnp.int32)
    i = jnp.int32(0x5F3759DF) - jnp.right_shift(i, 1)
    y = plsc.bitcast(i, jnp.float32)
    xh = x * 0.5
    for _ in range(3):
        y = y * (1.5 - xh * y * y)
    return y


def _body(idx3_hbm, wword_hbm, went_hbm, wattr_hbm, gb_hbm, bb_hbm, out_hbm,
          went_v, wattr_v, gb_v, bb_v,
          idx_v0, idx_v1, rows_v0, rows_v1,
          sem_i0, sem_i1, sem_g0, sem_g1, sem_o0, sem_o1,
          *, n_tokens):
    per_w = n_tokens // NW
    n_chunks = per_w // CHUNK
    wid = lax.axis_index("s") * NC + lax.axis_index("c")

    idx_v = (idx_v0, idx_v1)
    rows_v = (rows_v0, rows_v1)
    sem_i = (sem_i0, sem_i1)
    sem_g = (sem_g0, sem_g1)
    sem_o = (sem_o0, sem_o1)

    pltpu.sync_copy(went_hbm, went_v)
    pltpu.sync_copy(wattr_hbm, wattr_v)
    pltpu.sync_copy(gb_hbm, gb_v)
    pltpu.sync_copy(bb_hbm, bb_v)

    iota = lax.iota(jnp.int32, LANES)

    def chunk_off(ci):
        return wid * per_w + ci * CHUNK

    def issue_idx(ci, s):
        pltpu.async_copy(idx3_hbm.at[:, pl.ds(chunk_off(ci), CHUNK)],
                         idx_v[s], sem_i[s])

    def wait_idx(s):
        pltpu.make_async_copy(idx3_hbm.at[:, pl.ds(0, CHUNK)],
                              idx_v[s], sem_i[s]).wait()

    def issue_gather(s):
        pltpu.async_copy(wword_hbm.at[idx_v[s].at[0]], rows_v[s], sem_g[s])

    def wait_gather(s):
        pltpu.make_async_copy(wword_hbm.at[idx_v[s].at[0]],
                              rows_v[s], sem_g[s]).wait()

    def issue_out(ci, s):
        pltpu.async_copy(rows_v[s], out_hbm.at[pl.ds(chunk_off(ci), CHUNK)],
                         sem_o[s])

    def wait_out(s):
        pltpu.make_async_copy(rows_v[s], out_hbm.at[pl.ds(0, CHUNK)],
                              sem_o[s]).wait()

    def compute(s):
        rows = rows_v[s]
        eav = idx_v[s]

        def group_body(g):
            tvec = g * LANES + iota
            evec = eav[1, pl.ds(g * LANES, LANES)]
            avec = eav[2, pl.ds(g * LANES, LANES)]
            zero = jnp.zeros((LANES,), jnp.float32)
            ssum = zero
            ssq = zero
            svals = []
            for d in range(DIM):
                dsp = jnp.full((LANES,), d, jnp.int32)
                wv = plsc.load_gather(rows, [tvec, dsp])
                ev = plsc.load_gather(went_v, [evec, dsp])
                av = plsc.load_gather(wattr_v, [avec, dsp])
                sv = wv + ev + av
                svals.append(sv)
                ssum = ssum + sv
                ssq = ssq + sv * sv
            mean = ssum * (1.0 / DIM)
            var = ssq * (1.0 / DIM) - mean * mean
            rstd = _rsqrt(var + EPS)
            for d in range(DIM):
                dsp = jnp.full((LANES,), d, jnp.int32)
                a = rstd * gb_v[d, :]
                o = (svals[d] - mean) * a + bb_v[d, :]
                plsc.store_scatter(rows, [tvec, dsp], o)

        pl.loop(0, CHUNK // LANES)(group_body)

    issue_idx(0, 0)
    issue_idx(1, 1)
    wait_idx(0)
    issue_gather(0)

    def chunk_iter(ci, p):
        q = 1 - p

        @pl.when(ci < n_chunks - 1)
        def _prefetch():
            wait_idx(q)

            @pl.when(ci >= 1)
            def _drain_prev_out():
                wait_out(q)

            issue_gather(q)

        wait_gather(p)
        compute(p)
        issue_out(ci, p)

        @pl.when(ci < n_chunks - 2)
        def _prefetch_idx():
            issue_idx(ci + 2, p)

    def pair_body(base):
        chunk_iter(base, 0)
        chunk_iter(base + 1, 1)

    pl.loop(0, n_chunks, step=2)(pair_body)

    wait_out(0)
    wait_out(1)


def kernel(word, entity, attribute_key, Wword, Went, Wattr, gamma, beta):
    shape = word.shape
    n_tokens = shape[0] * shape[1]
    assert n_tokens % (NW * CHUNK) == 0
    assert (n_tokens // (NW * CHUNK)) % 2 == 0

    idx3 = jnp.stack([
        word.reshape(n_tokens).astype(jnp.int32),
        entity.reshape(n_tokens).astype(jnp.int32),
        attribute_key.reshape(n_tokens).astype(jnp.int32),
    ])
    gb = jnp.broadcast_to(gamma.astype(jnp.float32)[:, None], (DIM, LANES))
    bb = jnp.broadcast_to(beta.astype(jnp.float32)[:, None], (DIM, LANES))

    mesh = plsc.VectorSubcoreMesh(core_axis_name="c", subcore_axis_name="s")
    run = pl.kernel(
        functools.partial(_body, n_tokens=n_tokens),
        out_type=jax.ShapeDtypeStruct((n_tokens, DIM), jnp.float32),
        mesh=mesh,
        compiler_params=pltpu.CompilerParams(
            needs_layout_passes=False, use_tc_tiling_on_sc=False),
        scratch_types=[
            pltpu.VMEM((POS, DIM), jnp.float32),
            pltpu.VMEM((POS, DIM), jnp.float32),
            pltpu.VMEM((DIM, LANES), jnp.float32),
            pltpu.VMEM((DIM, LANES), jnp.float32),
            pltpu.VMEM((3, CHUNK), jnp.int32),
            pltpu.VMEM((3, CHUNK), jnp.int32),
            pltpu.VMEM((CHUNK, DIM), jnp.float32),
            pltpu.VMEM((CHUNK, DIM), jnp.float32),
            pltpu.SemaphoreType.DMA,
            pltpu.SemaphoreType.DMA,
            pltpu.SemaphoreType.DMA,
            pltpu.SemaphoreType.DMA,
            pltpu.SemaphoreType.DMA,
            pltpu.SemaphoreType.DMA,
        ],
    )
    out = run(idx3, Wword.astype(jnp.float32), Went.astype(jnp.float32),
              Wattr.astype(jnp.float32), gb, bb)
    return out.reshape(shape[0], shape[1], DIM)
